# baseline (speedup 1.0000x reference)
import functools

import jax
import jax.numpy as jnp
from jax.experimental import pallas as pl
from jax.experimental.pallas import tpu as pltpu

TOPK = 8


def _row_mean(xx):
    n = xx.shape[1]
    acc = xx[:, 0:128]
    for k in range(1, n // 128):
        acc = acc + xx[:, 128 * k:128 * (k + 1)]
    r = acc
    for k in range(1, 16):
        r = r + jnp.roll(acc, -8 * k, axis=1)
    c = r + jnp.roll(r, -4, axis=1)
    d = c + jnp.roll(c, -2, axis=1)
    e = d + jnp.roll(d, -1, axis=1)
    return e[:, 0:1] * (1.0 / n)


def _layer_norm_to(xx, g, bt, xn_ref):
    mu = _row_mean(xx)
    xc = xx - mu
    var = _row_mean(xc * xc)
    xn = xc / jnp.sqrt(var + 1e-5) * g + bt
    xn_ref[...] = xn.astype(jnp.bfloat16)


def _router_body(x_ref, g_ref, bt_ref, w1_ref, b1_ref, w2_ref, b2_ref,
                 ti_ref, tv_ref, ps_ref,
                 xn_ref, acc_ref,
                 *, n_chunks, n_tblocks, n_experts):
    t = pl.program_id(0)
    n = pl.program_id(1)

    ln_here = (n == 0) if n_chunks == 1 else (t == 0) & (n == 0)

    @pl.when(ln_here)
    def _first_ln():
        _layer_norm_to(x_ref[...], g_ref[...], bt_ref[...], xn_ref)

    @pl.when(n == 0)
    def _zero_acc():
        acc_ref[...] = jnp.zeros_like(acc_ref)

    h = jnp.dot(xn_ref[...], w1_ref[...],
                preferred_element_type=jnp.float32)
    h = jnp.maximum(h + b1_ref[...], 0.0).astype(jnp.bfloat16)
    acc_ref[...] += jnp.dot(h, w2_ref[...],
                            preferred_element_type=jnp.float32)

    @pl.when(n == n_chunks - 1)
    def _epilogue():
        logits = acc_ref[...] + b2_ref[...]
        m = jnp.max(logits, axis=1, keepdims=True)
        e = jnp.exp(logits - m)
        p = e / jnp.sum(e, axis=1, keepdims=True)
        ps_ref[...] = jnp.sum(p, axis=0, keepdims=True)[None]

        iota = jax.lax.broadcasted_iota(jnp.int32, p.shape, 1)
        v = p
        tvs, tis = [], []
        for _ in range(TOPK):
            mk = jnp.max(v, axis=1, keepdims=True)
            ik = jnp.min(jnp.where(v == mk, iota, n_experts), axis=1,
                         keepdims=True)
            tvs.append(mk)
            tis.append(ik)
            v = jnp.where(iota == ik, -1.0, v)
        tv = jnp.concatenate(tvs, axis=1)
        ti_ref[...] = jnp.concatenate(tis, axis=1)
        tv_ref[...] = tv / jnp.sum(tv, axis=1, keepdims=True)

        if n_chunks > 1:
            @pl.when(t < n_tblocks - 1)
            def _next_ln():
                _layer_norm_to(x_ref[...], g_ref[...], bt_ref[...], xn_ref)


def _aux_body(ps_ref, aux_ref, *, n_tokens, n_experts):
    s = jnp.sum(ps_ref[...], axis=(0, 1)).reshape(1, -1) / n_tokens
    aux_ref[...] = jnp.sum(s * jnp.log(s * n_experts + 1e-9), axis=1,
                           keepdims=True)


def kernel(x, ln_gamma, ln_beta, W1, b1, W2, b2):
    B, S, H = x.shape
    E = W2.shape[1]
    BS = B * S

    TB = 512 if BS % 512 == 0 else BS
    NC = 512 if H % 512 == 0 else H
    n_tblocks = BS // TB
    n_chunks = H // NC

    x2 = x.reshape(BS, H)
    g2 = ln_gamma.reshape(1, H)
    bt2 = ln_beta.reshape(1, H)
    b1_2 = b1.reshape(1, H)
    b2_2 = b2.reshape(1, E)

    body = functools.partial(_router_body, n_chunks=n_chunks,
                             n_tblocks=n_tblocks, n_experts=E)

    if n_chunks == 1:
        def x_index(t, n):
            return (t, 0)
    else:
        def x_index(t, n):
            return (jnp.minimum(t + n // (n_chunks - 1), n_tblocks - 1), 0)

    ti, tv, ps = pl.pallas_call(
        body,
        grid=(n_tblocks, n_chunks),
        in_specs=[
            pl.BlockSpec((TB, H), x_index),
            pl.BlockSpec((1, H), lambda t, n: (0, 0)),
            pl.BlockSpec((1, H), lambda t, n: (0, 0)),
            pl.BlockSpec((H, NC), lambda t, n: (0, n)),
            pl.BlockSpec((1, NC), lambda t, n: (0, n)),
            pl.BlockSpec((NC, E), lambda t, n: (n, 0)),
            pl.BlockSpec((1, E), lambda t, n: (0, 0)),
        ],
        out_specs=[
            pl.BlockSpec((TB, TOPK), lambda t, n: (t, 0)),
            pl.BlockSpec((TB, TOPK), lambda t, n: (t, 0)),
            pl.BlockSpec((1, 1, E), lambda t, n: (t, 0, 0)),
        ],
        out_shape=[
            jax.ShapeDtypeStruct((BS, TOPK), jnp.int32),
            jax.ShapeDtypeStruct((BS, TOPK), jnp.float32),
            jax.ShapeDtypeStruct((n_tblocks, 1, E), jnp.float32),
        ],
        scratch_shapes=[
            pltpu.VMEM((TB, H), jnp.bfloat16),
            pltpu.VMEM((TB, E), jnp.float32),
        ],
        compiler_params=pltpu.CompilerParams(
            dimension_semantics=("arbitrary", "arbitrary")),
    )(x2, g2, bt2, W1.astype(jnp.bfloat16), b1_2, W2.astype(jnp.bfloat16),
      b2_2)

    aux = pl.pallas_call(
        functools.partial(_aux_body, n_tokens=BS, n_experts=E),
        out_shape=jax.ShapeDtypeStruct((1, 1), jnp.float32),
    )(ps)

    return (ti.reshape(B, S, TOPK), tv.reshape(B, S, TOPK),
            aux.reshape(()))


# speedup vs baseline: 1.1579x; 1.1579x over previous
import functools

import jax
import jax.numpy as jnp
from jax.experimental import pallas as pl
from jax.experimental.pallas import tpu as pltpu

TOPK = 8


def _row_mean(xx):
    n = xx.shape[1]
    acc = xx[:, 0:128]
    for k in range(1, n // 128):
        acc = acc + xx[:, 128 * k:128 * (k + 1)]
    tr = acc.T
    s = tr[0:8]
    for k in range(1, 16):
        s = s + tr[8 * k:8 * (k + 1)]
    c = s[0:4] + s[4:8]
    d = c[0:2] + c[2:4]
    e = d[0:1] + d[1:2]
    return e.T * (1.0 / n)


def _layer_norm_to(xx, g, bt, xn_ref):
    mu = _row_mean(xx)
    xc = xx - mu
    var = _row_mean(xc * xc)
    xn = xc / jnp.sqrt(var + 1e-5) * g + bt
    xn_ref[...] = xn.astype(jnp.bfloat16)


def _router_body(x_ref, g_ref, bt_ref, w1_ref, b1_ref, w2_ref, b2_ref,
                 ti_ref, tv_ref, ps_ref,
                 xn_ref, acc_ref,
                 *, n_chunks, n_tblocks, n_experts):
    t = pl.program_id(0)
    n = pl.program_id(1)

    @pl.when(n == 0)
    def _prologue():
        _layer_norm_to(x_ref[...], g_ref[...], bt_ref[...], xn_ref)
        acc_ref[...] = jnp.zeros_like(acc_ref)

    h = jnp.dot(xn_ref[...], w1_ref[...],
                preferred_element_type=jnp.float32)
    h = jnp.maximum(h + b1_ref[...], 0.0).astype(jnp.bfloat16)
    acc_ref[...] += jnp.dot(h, w2_ref[...],
                            preferred_element_type=jnp.float32)

    @pl.when(n == n_chunks - 1)
    def _epilogue():
        logits = acc_ref[...] + b2_ref[...]
        m = jnp.max(logits, axis=1, keepdims=True)
        e = jnp.exp(logits - m)
        p = e / jnp.sum(e, axis=1, keepdims=True)
        ps_ref[...] = jnp.sum(p, axis=0, keepdims=True)[None]

        iota = jax.lax.broadcasted_iota(jnp.int32, p.shape, 1)
        v = p
        tvs, tis = [], []
        for _ in range(TOPK):
            mk = jnp.max(v, axis=1, keepdims=True)
            ik = jnp.min(jnp.where(v == mk, iota, n_experts), axis=1,
                         keepdims=True)
            tvs.append(mk)
            tis.append(ik)
            v = jnp.where(iota == ik, -1.0, v)
        tv = jnp.concatenate(tvs, axis=1)
        ti_ref[...] = jnp.concatenate(tis, axis=1)
        tv_ref[...] = tv / jnp.sum(tv, axis=1, keepdims=True)


def _aux_body(ps_ref, aux_ref, *, n_tokens, n_experts):
    s = jnp.sum(ps_ref[...], axis=(0, 1)).reshape(1, -1) / n_tokens
    aux_ref[...] = jnp.sum(s * jnp.log(s * n_experts + 1e-9), axis=1,
                           keepdims=True)


def kernel(x, ln_gamma, ln_beta, W1, b1, W2, b2):
    B, S, H = x.shape
    E = W2.shape[1]
    BS = B * S

    TB = 512 if BS % 512 == 0 else BS
    NC = 512 if H % 512 == 0 else H
    n_tblocks = BS // TB
    n_chunks = H // NC

    x2 = x.reshape(BS, H)
    g2 = ln_gamma.reshape(1, H)
    bt2 = ln_beta.reshape(1, H)
    b1_2 = b1.reshape(1, H)
    b2_2 = b2.reshape(1, E)

    body = functools.partial(_router_body, n_chunks=n_chunks,
                             n_tblocks=n_tblocks, n_experts=E)

    def x_index(t, n):
        return (t, 0)

    ti, tv, ps = pl.pallas_call(
        body,
        grid=(n_tblocks, n_chunks),
        in_specs=[
            pl.BlockSpec((TB, H), x_index),
            pl.BlockSpec((1, H), lambda t, n: (0, 0)),
            pl.BlockSpec((1, H), lambda t, n: (0, 0)),
            pl.BlockSpec((H, NC), lambda t, n: (0, n)),
            pl.BlockSpec((1, NC), lambda t, n: (0, n)),
            pl.BlockSpec((NC, E), lambda t, n: (n, 0)),
            pl.BlockSpec((1, E), lambda t, n: (0, 0)),
        ],
        out_specs=[
            pl.BlockSpec((TB, TOPK), lambda t, n: (t, 0)),
            pl.BlockSpec((TB, TOPK), lambda t, n: (t, 0)),
            pl.BlockSpec((1, 1, E), lambda t, n: (t, 0, 0)),
        ],
        out_shape=[
            jax.ShapeDtypeStruct((BS, TOPK), jnp.int32),
            jax.ShapeDtypeStruct((BS, TOPK), jnp.float32),
            jax.ShapeDtypeStruct((n_tblocks, 1, E), jnp.float32),
        ],
        scratch_shapes=[
            pltpu.VMEM((TB, H), jnp.bfloat16),
            pltpu.VMEM((TB, E), jnp.float32),
        ],
        compiler_params=pltpu.CompilerParams(
            dimension_semantics=("arbitrary", "arbitrary")),
    )(x2, g2, bt2, W1.astype(jnp.bfloat16), b1_2, W2.astype(jnp.bfloat16),
      b2_2)

    aux = pl.pallas_call(
        functools.partial(_aux_body, n_tokens=BS, n_experts=E),
        out_shape=jax.ShapeDtypeStruct((1, 1), jnp.float32),
    )(ps)

    return (ti.reshape(B, S, TOPK), tv.reshape(B, S, TOPK),
            aux.reshape(()))


# speedup vs baseline: 1.2043x; 1.0401x over previous
import functools

import jax
import jax.numpy as jnp
from jax.experimental import pallas as pl
from jax.experimental.pallas import tpu as pltpu

TOPK = 8


def _row_mean(xx):
    n = xx.shape[1]
    acc = xx[:, 0:128]
    for k in range(1, n // 128):
        acc = acc + xx[:, 128 * k:128 * (k + 1)]
    tr = acc.T
    s = tr[0:8]
    for k in range(1, 16):
        s = s + tr[8 * k:8 * (k + 1)]
    c = s[0:4] + s[4:8]
    d = c[0:2] + c[2:4]
    e = d[0:1] + d[1:2]
    return e.T * (1.0 / n)


def _layer_norm_to(xx, g, bt, xn_ref):
    mu = _row_mean(xx)
    xc = xx - mu
    var = _row_mean(xc * xc)
    xn = xc / jnp.sqrt(var + 1e-5) * g + bt
    xn_ref[...] = xn.astype(jnp.bfloat16)


def _router_body(x_ref, g_ref, bt_ref, w1_ref, b1_ref, w2_ref, b2_ref,
                 ti_ref, tv_ref, ps_ref,
                 xn_ref, acc_ref,
                 *, n_chunks, n_tblocks, n_experts):
    t = pl.program_id(0)
    n = pl.program_id(1)

    @pl.when(n == 0)
    def _prologue():
        _layer_norm_to(x_ref[...], g_ref[...], bt_ref[...], xn_ref)
        acc_ref[...] = jnp.zeros_like(acc_ref)

    h = jnp.dot(xn_ref[...], w1_ref[...],
                preferred_element_type=jnp.float32)
    h = jnp.maximum(h + b1_ref[...], 0.0).astype(jnp.bfloat16)
    acc_ref[...] += jnp.dot(h, w2_ref[...],
                            preferred_element_type=jnp.float32)

    @pl.when(n == n_chunks - 1)
    def _epilogue():
        logits = acc_ref[...] + b2_ref[...]
        m = jnp.max(logits, axis=1, keepdims=True)
        e = jnp.exp(logits - m)
        p = e / jnp.sum(e, axis=1, keepdims=True)
        ps_ref[...] = jnp.sum(p, axis=0, keepdims=True)[None]

        iota = jax.lax.broadcasted_iota(jnp.int32, p.shape, 1)
        v = p
        tvs, tis = [], []
        for _ in range(TOPK):
            mk = jnp.max(v, axis=1, keepdims=True)
            ik = jnp.min(jnp.where(v == mk, iota, n_experts), axis=1,
                         keepdims=True)
            tvs.append(mk)
            tis.append(ik)
            v = jnp.where(iota == ik, -1.0, v)
        tv = jnp.concatenate(tvs, axis=1)
        ti_ref[...] = jnp.concatenate(tis, axis=1)
        tv_ref[...] = tv / jnp.sum(tv, axis=1, keepdims=True)


def _aux_body(ps_ref, aux_ref, *, n_tokens, n_experts):
    s = jnp.sum(ps_ref[...], axis=(0, 1)).reshape(1, -1) / n_tokens
    aux_ref[...] = jnp.sum(s * jnp.log(s * n_experts + 1e-9), axis=1,
                           keepdims=True)


def kernel(x, ln_gamma, ln_beta, W1, b1, W2, b2):
    B, S, H = x.shape
    E = W2.shape[1]
    BS = B * S

    TB = 512 if BS % 512 == 0 else BS
    NC = 1024 if H % 1024 == 0 else H
    n_tblocks = BS // TB
    n_chunks = H // NC

    x2 = x.reshape(BS, H)
    g2 = ln_gamma.reshape(1, H)
    bt2 = ln_beta.reshape(1, H)
    b1_2 = b1.reshape(1, H)
    b2_2 = b2.reshape(1, E)

    body = functools.partial(_router_body, n_chunks=n_chunks,
                             n_tblocks=n_tblocks, n_experts=E)

    def x_index(t, n):
        return (t, 0)

    ti, tv, ps = pl.pallas_call(
        body,
        grid=(n_tblocks, n_chunks),
        in_specs=[
            pl.BlockSpec((TB, H), x_index),
            pl.BlockSpec((1, H), lambda t, n: (0, 0)),
            pl.BlockSpec((1, H), lambda t, n: (0, 0)),
            pl.BlockSpec((H, NC), lambda t, n: (0, n)),
            pl.BlockSpec((1, NC), lambda t, n: (0, n)),
            pl.BlockSpec((NC, E), lambda t, n: (n, 0)),
            pl.BlockSpec((1, E), lambda t, n: (0, 0)),
        ],
        out_specs=[
            pl.BlockSpec((TB, TOPK), lambda t, n: (t, 0)),
            pl.BlockSpec((TB, TOPK), lambda t, n: (t, 0)),
            pl.BlockSpec((1, 1, E), lambda t, n: (t, 0, 0)),
        ],
        out_shape=[
            jax.ShapeDtypeStruct((BS, TOPK), jnp.int32),
            jax.ShapeDtypeStruct((BS, TOPK), jnp.float32),
            jax.ShapeDtypeStruct((n_tblocks, 1, E), jnp.float32),
        ],
        scratch_shapes=[
            pltpu.VMEM((TB, H), jnp.bfloat16),
            pltpu.VMEM((TB, E), jnp.float32),
        ],
        compiler_params=pltpu.CompilerParams(
            dimension_semantics=("arbitrary", "arbitrary")),
    )(x2, g2, bt2, W1.astype(jnp.bfloat16), b1_2, W2.astype(jnp.bfloat16),
      b2_2)

    aux = pl.pallas_call(
        functools.partial(_aux_body, n_tokens=BS, n_experts=E),
        out_shape=jax.ShapeDtypeStruct((1, 1), jnp.float32),
    )(ps)

    return (ti.reshape(B, S, TOPK), tv.reshape(B, S, TOPK),
            aux.reshape(()))
